# baseline (speedup 1.0000x reference)
import functools

import jax
import jax.numpy as jnp
from jax import lax
from jax.experimental import pallas as pl
from jax.experimental.pallas import tpu as pltpu
from jax.experimental.pallas import tpu_sc as plsc

_B = 16384
_D = 64
_AR = 20
_NN = 5
_NG = _NN + 1
_NC = 2
_NS = 16
_NW = _NC * _NS
_SPW = _B // _NW
_CS = 8
_NCH = _SPW // _CS
_GPC = _CS * _NG
_RPC = _GPC * _AR
_WROWS = _RPC
_NWIN = _RPC // _WROWS
_LANES = 16
_NQ = _D // _LANES


def _sc_partials(xp_flat, xn_flat, emb):
    mesh = plsc.VectorSubcoreMesh(core_axis_name="c", subcore_axis_name="s")

    @functools.partial(
        pl.kernel,
        out_type=jax.ShapeDtypeStruct((_B * _NG * _LANES // 128, 128), jnp.float32),
        mesh=mesh,
        compiler_params=pltpu.CompilerParams(use_tc_tiling_on_sc=False),
        scratch_types=[
            pltpu.VMEM((_RPC,), jnp.int32),
            pltpu.VMEM((_RPC,), jnp.int32),
            pltpu.VMEM((_RPC, _D), jnp.float32),
            pltpu.VMEM((_RPC, _D), jnp.float32),
            pltpu.VMEM((_GPC * _LANES // 128, 128), jnp.float32),
            pltpu.VMEM((_GPC * _LANES // 128, 128), jnp.float32),
            pltpu.SemaphoreType.DMA,
            pltpu.SemaphoreType.DMA,
            pltpu.SemaphoreType.DMA,
            pltpu.SemaphoreType.DMA,
            pltpu.SemaphoreType.DMA,
            pltpu.SemaphoreType.DMA,
        ],
    )
    def k(xp_hbm, xn_hbm, emb_hbm, out_hbm,
          idx_v0, idx_v1, rows_v0, rows_v1, part_v0, part_v1,
          sem0, sem1, psem0, psem1, isem0, isem1):
        wid = lax.axis_index("s") * _NC + lax.axis_index("c")
        idx_b = (idx_v0, idx_v1)
        rows_b = (rows_v0, rows_v1)
        part_b = (part_v0, part_v1)
        sem_b = (sem0, sem1)
        psem_b = (psem0, psem1)
        isem_b = (isem0, isem1)

        def stage_idx(ch, b):
            pltpu.async_copy(
                xp_hbm.at[pl.ds(wid * (_SPW * _AR) + ch * (_CS * _AR), _CS * _AR)],
                idx_b[b].at[pl.ds(0, _CS * _AR)],
                isem_b[b])
            pltpu.async_copy(
                xn_hbm.at[pl.ds(wid * (_SPW * _NN * _AR) + ch * (_CS * _NN * _AR),
                                _CS * _NN * _AR)],
                idx_b[b].at[pl.ds(_CS * _AR, _CS * _NN * _AR)],
                isem_b[b])

        def wait_idx(b):
            pltpu.make_async_copy(
                xp_hbm.at[pl.ds(0, _RPC)], idx_b[b], isem_b[b]).wait()

        def fire_gathers(b):
            for j in range(_NWIN):
                pltpu.async_copy(
                    emb_hbm.at[idx_b[b].at[pl.ds(j * _WROWS, _WROWS)]],
                    rows_b[b].at[pl.ds(j * _WROWS, _WROWS)],
                    sem_b[b])

        def drain_rows(b):
            pltpu.make_async_copy(
                emb_hbm.at[pl.ds(0, _RPC)], rows_b[b], sem_b[b]).wait()

        for b in range(2):
            stage_idx(b, b)
            wait_idx(b)
            fire_gathers(b)

        @pl.loop(0, _NCH // 2)
        def _pair(p):
            for b in range(2):
                ch = 2 * p + b
                drain_rows(b)

                @pl.when(ch < _NCH - 2)
                def _():
                    stage_idx(ch + 2, b)

                @pl.when(ch >= 2)
                def _():
                    pltpu.make_async_copy(
                        out_hbm.at[pl.ds(0, _GPC * _LANES // 128), :], part_b[b],
                        psem_b[b]).wait()

                @pl.loop(0, _CS)
                def _sample(sl):
                    for g in range(_NG):
                        if g == 0:
                            base = sl * _AR
                        else:
                            base = _CS * _AR + sl * (_NN * _AR) + (g - 1) * _AR
                        acc = [rows_b[b][pl.ds(base, 1), pl.ds(q * _LANES, _LANES)]
                               for q in range(_NQ)]
                        for r in range(1, _AR):
                            for q in range(_NQ):
                                acc[q] += rows_b[b][pl.ds(base + r, 1),
                                                    pl.ds(q * _LANES, _LANES)]
                        sq = acc[0] * acc[0]
                        for q in range(1, _NQ):
                            sq += acc[q] * acc[q]
                        gi = sl * _NG + g
                        part_b[b][pl.ds(gi // 8, 1),
                                  pl.ds((gi % 8) * _LANES, _LANES)] = sq

                pltpu.async_copy(
                    part_b[b],
                    out_hbm.at[pl.ds((wid * (_SPW * _NG) + ch * _GPC) * _LANES
                                     // 128, _GPC * _LANES // 128), :],
                    psem_b[b])

                @pl.when(ch < _NCH - 2)
                def _():
                    wait_idx(b)
                    fire_gathers(b)

        for b in range(2):
            pltpu.make_async_copy(
                out_hbm.at[pl.ds(0, _GPC * _LANES // 128), :], part_b[b],
                psem_b[b]).wait()

    return k(xp_flat, xn_flat, emb)


def _tc_score(parts):
    def body(p_ref, o_ref):
        v = p_ref[...]
        for sh in (1, 2, 4, 8):
            v = v + jnp.concatenate([v[:, sh:], v[:, :sh]], axis=1)
        r = lax.broadcasted_iota(jnp.int32, v.shape, 0)
        c = lax.broadcasted_iota(jnp.int32, v.shape, 1)
        gi = r * (v.shape[1] // _LANES) + c // _LANES
        is_start = (c % _LANES) == 0
        y = jnp.where((gi % _NG) == 0, v, jnp.reciprocal(v))
        val = jnp.log(jnp.tanh(y))
        val = jnp.where(is_start, val, 0.0)
        o_ref[...] = (jnp.sum(val) * (1.0 / _B)).reshape(1, 1)

    return pl.pallas_call(
        body,
        out_shape=jax.ShapeDtypeStruct((1, 1), jnp.float32),
    )(parts)


def kernel(x_pos, x_neg, emb):
    xp = x_pos.reshape(-1)
    xn = x_neg.reshape(-1)
    parts = _sc_partials(xp, xn, emb)
    return _tc_score(parts).reshape(())


# speedup vs baseline: 1.0013x; 1.0013x over previous
import functools

import jax
import jax.numpy as jnp
from jax import lax
from jax.experimental import pallas as pl
from jax.experimental.pallas import tpu as pltpu
from jax.experimental.pallas import tpu_sc as plsc

_B = 16384
_D = 64
_AR = 20
_NN = 5
_NG = _NN + 1
_NC = 2
_NS = 16
_NW = _NC * _NS
_SPW = _B // _NW
_CS = 8
_NCH = _SPW // _CS
_GPC = _CS * _NG
_RPC = _GPC * _AR
_WROWS = _NG * _AR
_NWIN = _RPC // _WROWS
_LANES = 16
_NQ = _D // _LANES


def _sc_partials(xp_flat, xn_flat, emb):
    mesh = plsc.VectorSubcoreMesh(core_axis_name="c", subcore_axis_name="s")

    @functools.partial(
        pl.kernel,
        out_type=jax.ShapeDtypeStruct((_B * _NG * _LANES // 128, 128), jnp.float32),
        mesh=mesh,
        compiler_params=pltpu.CompilerParams(use_tc_tiling_on_sc=False),
        scratch_types=[
            pltpu.VMEM((_RPC,), jnp.int32),
            pltpu.VMEM((_RPC,), jnp.int32),
            pltpu.VMEM((_RPC, _D), jnp.float32),
            pltpu.VMEM((_RPC, _D), jnp.float32),
            pltpu.VMEM((_GPC * _LANES // 128, 128), jnp.float32),
            pltpu.VMEM((_GPC * _LANES // 128, 128), jnp.float32),
            pltpu.SemaphoreType.DMA,
            pltpu.SemaphoreType.DMA,
            pltpu.SemaphoreType.DMA,
            pltpu.SemaphoreType.DMA,
            pltpu.SemaphoreType.DMA,
            pltpu.SemaphoreType.DMA,
        ],
    )
    def k(xp_hbm, xn_hbm, emb_hbm, out_hbm,
          idx_v0, idx_v1, rows_v0, rows_v1, part_v0, part_v1,
          sem0, sem1, psem0, psem1, isem0, isem1):
        wid = lax.axis_index("s") * _NC + lax.axis_index("c")
        idx_b = (idx_v0, idx_v1)
        rows_b = (rows_v0, rows_v1)
        part_b = (part_v0, part_v1)
        sem_b = (sem0, sem1)
        psem_b = (psem0, psem1)
        isem_b = (isem0, isem1)

        def stage_idx(ch, b):
            pltpu.async_copy(
                xp_hbm.at[pl.ds(wid * (_SPW * _AR) + ch * (_CS * _AR), _CS * _AR)],
                idx_b[b].at[pl.ds(0, _CS * _AR)],
                isem_b[b])
            pltpu.async_copy(
                xn_hbm.at[pl.ds(wid * (_SPW * _NN * _AR) + ch * (_CS * _NN * _AR),
                                _CS * _NN * _AR)],
                idx_b[b].at[pl.ds(_CS * _AR, _CS * _NN * _AR)],
                isem_b[b])

        def wait_idx(b):
            pltpu.make_async_copy(
                xp_hbm.at[pl.ds(0, _RPC)], idx_b[b], isem_b[b]).wait()

        def fire_gathers(b):
            for j in range(_NWIN):
                pltpu.async_copy(
                    emb_hbm.at[idx_b[b].at[pl.ds(j * _WROWS, _WROWS)]],
                    rows_b[b].at[pl.ds(j * _WROWS, _WROWS)],
                    sem_b[b])

        def drain_rows(b):
            pltpu.make_async_copy(
                emb_hbm.at[pl.ds(0, _RPC)], rows_b[b], sem_b[b]).wait()

        for b in range(2):
            stage_idx(b, b)
            wait_idx(b)
            fire_gathers(b)

        @pl.loop(0, _NCH // 2)
        def _pair(p):
            for b in range(2):
                ch = 2 * p + b
                drain_rows(b)

                @pl.when(ch < _NCH - 2)
                def _():
                    stage_idx(ch + 2, b)

                @pl.when(ch >= 2)
                def _():
                    pltpu.make_async_copy(
                        out_hbm.at[pl.ds(0, _GPC * _LANES // 128), :], part_b[b],
                        psem_b[b]).wait()

                @pl.loop(0, _CS)
                def _sample(sl):
                    for g in range(_NG):
                        if g == 0:
                            base = sl * _AR
                        else:
                            base = _CS * _AR + sl * (_NN * _AR) + (g - 1) * _AR
                        acc = [rows_b[b][pl.ds(base, 1), pl.ds(q * _LANES, _LANES)]
                               for q in range(_NQ)]
                        for r in range(1, _AR):
                            for q in range(_NQ):
                                acc[q] += rows_b[b][pl.ds(base + r, 1),
                                                    pl.ds(q * _LANES, _LANES)]
                        sq = acc[0] * acc[0]
                        for q in range(1, _NQ):
                            sq += acc[q] * acc[q]
                        gi = sl * _NG + g
                        part_b[b][pl.ds(gi // 8, 1),
                                  pl.ds((gi % 8) * _LANES, _LANES)] = sq

                pltpu.async_copy(
                    part_b[b],
                    out_hbm.at[pl.ds((wid * (_SPW * _NG) + ch * _GPC) * _LANES
                                     // 128, _GPC * _LANES // 128), :],
                    psem_b[b])

                @pl.when(ch < _NCH - 2)
                def _():
                    wait_idx(b)
                    fire_gathers(b)

        for b in range(2):
            pltpu.make_async_copy(
                out_hbm.at[pl.ds(0, _GPC * _LANES // 128), :], part_b[b],
                psem_b[b]).wait()

    return k(xp_flat, xn_flat, emb)


def _tc_score(parts):
    def body(p_ref, o_ref):
        v = p_ref[...]
        for sh in (1, 2, 4, 8):
            v = v + jnp.concatenate([v[:, sh:], v[:, :sh]], axis=1)
        r = lax.broadcasted_iota(jnp.int32, v.shape, 0)
        c = lax.broadcasted_iota(jnp.int32, v.shape, 1)
        gi = r * (v.shape[1] // _LANES) + c // _LANES
        is_start = (c % _LANES) == 0
        y = jnp.where((gi % _NG) == 0, v, jnp.reciprocal(v))
        val = jnp.log(jnp.tanh(y))
        val = jnp.where(is_start, val, 0.0)
        o_ref[...] = (jnp.sum(val) * (1.0 / _B)).reshape(1, 1)

    return pl.pallas_call(
        body,
        out_shape=jax.ShapeDtypeStruct((1, 1), jnp.float32),
    )(parts)


def kernel(x_pos, x_neg, emb):
    xp = x_pos.reshape(-1)
    xn = x_neg.reshape(-1)
    parts = _sc_partials(xp, xn, emb)
    return _tc_score(parts).reshape(())
